# baseline (speedup 1.0000x reference)
import functools

import jax
import jax.numpy as jnp
from jax import lax
from jax.experimental import pallas as pl
from jax.experimental.pallas import tpu as pltpu
from jax.experimental.pallas import tpu_sc as plsc

_MIN_STATES = 4
_MAX_STATES = 64
_CHUNK = 512
_NBUF = 8


def _imp_body(sb_ref, Wi1_ref, bi1_ref, Wi2_ref, bi2_ref, temp_ref, imp_ref):
    def dot_t(a, w):
        return lax.dot_general(a, w, (((1,), (1,)), ((), ())),
                               preferred_element_type=jnp.float32)

    hi = jax.nn.relu(dot_t(sb_ref[:, :], Wi1_ref[:, :]) + bi1_ref[:][None, :])
    logits = dot_t(hi, jnp.broadcast_to(Wi2_ref[:, :], (16, Wi2_ref.shape[1])))
    logits = logits + bi2_ref[0]
    temp = jnp.maximum(jnp.abs(temp_ref[0]), 0.1)
    imp_ref[...] = jax.nn.softmax(logits / temp, axis=0)


def _imp_call(state_bank, Wi1, bi1, Wi2, bi2, temperature):
    vmem = pl.BlockSpec(memory_space=pltpu.VMEM)
    smem = pl.BlockSpec(memory_space=pltpu.SMEM)
    return pl.pallas_call(
        _imp_body,
        in_specs=[vmem, vmem, vmem, vmem, smem, smem],
        out_specs=vmem,
        out_shape=jax.ShapeDtypeStruct((_MAX_STATES, 16), jnp.float32),
    )(state_bank, Wi1, bi1, Wi2, bi2, temperature)


def _sc_rank_call(imp2d):
    mesh = plsc.VectorSubcoreMesh(core_axis_name="c", subcore_axis_name="s")
    n_vregs = _MAX_STATES // 16

    @functools.partial(
        pl.kernel,
        mesh=mesh,
        out_type=jax.ShapeDtypeStruct((1, _MAX_STATES), jnp.int32),
        scratch_types=[
            pltpu.VMEM((_MAX_STATES, 16), jnp.float32),
            pltpu.VMEM((1, _MAX_STATES), jnp.int32),
        ],
    )
    def k(imp_hbm, rank_hbm, imp_v, rank_v):
        wid = lax.axis_index("s") * 2 + lax.axis_index("c")

        @pl.when(wid == 0)
        def _():
            pltpu.sync_copy(imp_hbm, imp_v)
            iota = lax.iota(jnp.int32, 16)

            def build(a):
                def lbody(l, v):
                    return jnp.where(iota == l, imp_v[16 * a + l, :], v)
                return lax.fori_loop(1, 16, lbody, imp_v[16 * a, :])

            vs = [build(a) for a in range(n_vregs)]
            gids = [iota + 16 * a for a in range(n_vregs)]

            def jbody(j, ranks):
                vj = imp_v[j, :]
                out = []
                for a in range(n_vregs):
                    cmp = (vj > vs[a]) | ((vj == vs[a]) & (j < gids[a]))
                    out.append(ranks[a] + jnp.where(cmp, 1, 0).astype(jnp.int32))
                return tuple(out)

            zero = jnp.zeros((16,), jnp.int32)
            ranks = lax.fori_loop(0, _MAX_STATES, jbody,
                                  tuple(zero for _ in range(n_vregs)))
            for a in range(n_vregs):
                rank_v[0, pl.ds(16 * a, 16)] = ranks[a]
            pltpu.sync_copy(rank_v, rank_hbm)

    return k(imp2d)


def _make_main_body(B, S, D):
    nchunks = (B * S) // _CHUNK
    chunks_per_batch = S // _CHUNK

    def body(xf_ref, sb_ref, W1_ref, b1_ref, W2_ref, b2_ref, W3_ref, b3_ref,
             alloc_ref, ns_ref, buf_ref, sems):
        def start(i, slot):
            pltpu.make_async_copy(
                xf_ref.at[pl.ds(i * _CHUNK, _CHUNK), :],
                buf_ref.at[slot], sems.at[slot]).start()

        def wait(slot):
            pltpu.make_async_copy(
                xf_ref.at[pl.ds(0, _CHUNK), :],
                buf_ref.at[slot], sems.at[slot]).wait()

        for slot in range(_NBUF):
            start(slot, slot)

        rows8 = jax.lax.broadcasted_iota(jnp.int32, (8, 1), 0)

        def round_body(r, acc):
            for slot in range(_NBUF):
                i = r * _NBUF + slot
                wait(slot)

                @pl.when(i + _NBUF < nchunks)
                def _():
                    start(i + _NBUF, slot)

                partial = jnp.sum(buf_ref[slot], axis=0)
                onehot = (rows8 == i // chunks_per_batch).astype(jnp.float32)
                acc = acc + onehot * partial[None, :]
            return acc

        acc = lax.fori_loop(0, nchunks // _NBUF, round_body,
                            jnp.zeros((8, D), jnp.float32))

        def dot_t(a, w):
            return lax.dot_general(a, w, (((1,), (1,)), ((), ())),
                                   preferred_element_type=jnp.float32)

        pooled = acc[:B, :] * (1.0 / S)
        h = jax.nn.relu(dot_t(pooled, W1_ref[:, :]) + b1_ref[:][None, :])
        h = jax.nn.relu(dot_t(h, W2_ref[:, :]) + b2_ref[:][None, :])
        z = dot_t(h, jnp.broadcast_to(W3_ref[:, :], (16, W3_ref.shape[1])))
        complexity = jax.nn.sigmoid(z + b3_ref[0])
        ns_ref[...] = jnp.clip(
            jnp.round(_MIN_STATES + complexity * (_MAX_STATES - _MIN_STATES)),
            _MIN_STATES, _MAX_STATES).astype(jnp.int32)

        alloc_ref[...] = jnp.broadcast_to(sb_ref[:, :][None, :, :],
                                          alloc_ref.shape)

    return body


def _main_call(x, state_bank, W1, b1, W2, b2, W3, b3):
    B, S, D = x.shape
    xf = x.reshape(B * S, D)

    vmem = pl.BlockSpec(memory_space=pltpu.VMEM)
    smem = pl.BlockSpec(memory_space=pltpu.SMEM)

    out_shape = (
        jax.ShapeDtypeStruct((B, _MAX_STATES, state_bank.shape[1]), jnp.float32),
        jax.ShapeDtypeStruct((B, 16), jnp.int32),
    )
    return pl.pallas_call(
        _make_main_body(B, S, D),
        in_specs=[pl.BlockSpec(memory_space=pl.ANY),
                  vmem, vmem, vmem, vmem, vmem, vmem, smem],
        out_specs=(vmem, vmem),
        out_shape=out_shape,
        scratch_shapes=[
            pltpu.VMEM((_NBUF, _CHUNK, D), jnp.float32),
            pltpu.SemaphoreType.DMA((_NBUF,)),
        ],
        compiler_params=pltpu.CompilerParams(
            vmem_limit_bytes=100 * 1024 * 1024),
    )(xf, state_bank, W1, b1, W2, b2, W3, b3)


def _combine_body(rank_ref, ns_ref, mask_ref):
    B = ns_ref.shape[0]
    reps = _MAX_STATES // ns_ref.shape[1]
    ns64 = jnp.concatenate([ns_ref[:, :]] * reps, axis=1)
    rank4 = jnp.broadcast_to(rank_ref[:, :], (B, _MAX_STATES))
    mask_ref[...] = jnp.where(rank4 < ns64, 1, 0).astype(jnp.int32)


def _combine_call(rankT, ns2):
    vmem = pl.BlockSpec(memory_space=pltpu.VMEM)
    return pl.pallas_call(
        _combine_body,
        in_specs=[vmem, vmem],
        out_specs=vmem,
        out_shape=jax.ShapeDtypeStruct((ns2.shape[0], _MAX_STATES), jnp.int32),
    )(rankT, ns2)


def kernel(x, state_bank, W1, b1, W2, b2, W3, b3, Wi1, bi1, Wi2, bi2,
           temperature):
    imp2d = _imp_call(state_bank, Wi1, bi1, Wi2, bi2, temperature)
    rankT = _sc_rank_call(imp2d)
    alloc, ns2 = _main_call(x, state_bank, W1, b1, W2, b2, W3, b3)
    mask_i32 = _combine_call(rankT, ns2)
    return alloc, mask_i32.astype(jnp.bool_)


# speedup vs baseline: 1.0143x; 1.0143x over previous
import functools

import jax
import jax.numpy as jnp
from jax import lax
from jax.experimental import pallas as pl
from jax.experimental.pallas import tpu as pltpu
from jax.experimental.pallas import tpu_sc as plsc

_MIN_STATES = 4
_MAX_STATES = 64
_CHUNK = 512
_NBUF = 8


def _imp_body(sb_ref, Wi1_ref, bi1_ref, Wi2_ref, bi2_ref, temp_ref, imp_ref):
    def dot_t(a, w):
        return lax.dot_general(a, w, (((1,), (1,)), ((), ())),
                               preferred_element_type=jnp.float32)

    hi = jax.nn.relu(dot_t(sb_ref[:, :], Wi1_ref[:, :]) + bi1_ref[:][None, :])
    logits = dot_t(hi, jnp.broadcast_to(Wi2_ref[:, :], (16, Wi2_ref.shape[1])))
    logits = logits + bi2_ref[0]
    temp = jnp.maximum(jnp.abs(temp_ref[0]), 0.1)
    imp_ref[...] = jax.nn.softmax(logits / temp, axis=0)


def _imp_call(state_bank, Wi1, bi1, Wi2, bi2, temperature):
    vmem = pl.BlockSpec(memory_space=pltpu.VMEM)
    smem = pl.BlockSpec(memory_space=pltpu.SMEM)
    return pl.pallas_call(
        _imp_body,
        in_specs=[vmem, vmem, vmem, vmem, smem, smem],
        out_specs=vmem,
        out_shape=jax.ShapeDtypeStruct((_MAX_STATES, 16), jnp.float32),
    )(state_bank, Wi1, bi1, Wi2, bi2, temperature)


def _sc_rank_call(imp2d):
    mesh = plsc.VectorSubcoreMesh(core_axis_name="c", subcore_axis_name="s",
                                  num_cores=1)
    n_vregs = _MAX_STATES // 16

    @functools.partial(
        pl.kernel,
        mesh=mesh,
        out_type=jax.ShapeDtypeStruct((1, _MAX_STATES), jnp.int32),
        scratch_types=[
            pltpu.VMEM((_MAX_STATES, 16), jnp.float32),
            pltpu.VMEM((1, _MAX_STATES), jnp.int32),
        ],
    )
    def k(imp_hbm, rank_hbm, imp_v, rank_v):
        wid = lax.axis_index("s") + lax.axis_index("c")

        @pl.when(wid == 0)
        def _():
            pltpu.sync_copy(imp_hbm, imp_v)
            iota = lax.iota(jnp.int32, 16)

            def build(a):
                def lbody(l, v):
                    return jnp.where(iota == l, imp_v[16 * a + l, :], v)
                return lax.fori_loop(1, 16, lbody, imp_v[16 * a, :])

            vs = [build(a) for a in range(n_vregs)]
            gids = [iota + 16 * a for a in range(n_vregs)]

            def jbody(j, ranks):
                vj = imp_v[j, :]
                out = []
                for a in range(n_vregs):
                    cmp = (vj > vs[a]) | ((vj == vs[a]) & (j < gids[a]))
                    out.append(ranks[a] + jnp.where(cmp, 1, 0).astype(jnp.int32))
                return tuple(out)

            zero = jnp.zeros((16,), jnp.int32)
            ranks = lax.fori_loop(0, _MAX_STATES, jbody,
                                  tuple(zero for _ in range(n_vregs)))
            for a in range(n_vregs):
                rank_v[0, pl.ds(16 * a, 16)] = ranks[a]
            pltpu.sync_copy(rank_v, rank_hbm)

    return k(imp2d)


def _make_main_body(B, S, D):
    nchunks = (B * S) // _CHUNK
    chunks_per_batch = S // _CHUNK

    def body(xf_ref, sb_ref, W1_ref, b1_ref, W2_ref, b2_ref, W3_ref, b3_ref,
             alloc_ref, ns_ref, buf_ref, sems):
        def start(i, slot):
            pltpu.make_async_copy(
                xf_ref.at[pl.ds(i * _CHUNK, _CHUNK), :],
                buf_ref.at[slot], sems.at[slot]).start()

        def wait(slot):
            pltpu.make_async_copy(
                xf_ref.at[pl.ds(0, _CHUNK), :],
                buf_ref.at[slot], sems.at[slot]).wait()

        for slot in range(_NBUF):
            start(slot, slot)

        rows8 = jax.lax.broadcasted_iota(jnp.int32, (8, 1), 0)

        def round_body(r, acc):
            for slot in range(_NBUF):
                i = r * _NBUF + slot
                wait(slot)

                @pl.when(i + _NBUF < nchunks)
                def _():
                    start(i + _NBUF, slot)

                partial = jnp.sum(buf_ref[slot], axis=0)
                onehot = (rows8 == i // chunks_per_batch).astype(jnp.float32)
                acc = acc + onehot * partial[None, :]
            return acc

        acc = lax.fori_loop(0, nchunks // _NBUF, round_body,
                            jnp.zeros((8, D), jnp.float32))

        def dot_t(a, w):
            return lax.dot_general(a, w, (((1,), (1,)), ((), ())),
                                   preferred_element_type=jnp.float32)

        pooled = acc[:B, :] * (1.0 / S)
        h = jax.nn.relu(dot_t(pooled, W1_ref[:, :]) + b1_ref[:][None, :])
        h = jax.nn.relu(dot_t(h, W2_ref[:, :]) + b2_ref[:][None, :])
        z = dot_t(h, jnp.broadcast_to(W3_ref[:, :], (16, W3_ref.shape[1])))
        complexity = jax.nn.sigmoid(z + b3_ref[0])
        ns_ref[...] = jnp.clip(
            jnp.round(_MIN_STATES + complexity * (_MAX_STATES - _MIN_STATES)),
            _MIN_STATES, _MAX_STATES).astype(jnp.int32)

        alloc_ref[...] = jnp.broadcast_to(sb_ref[:, :][None, :, :],
                                          alloc_ref.shape)

    return body


def _main_call(x, state_bank, W1, b1, W2, b2, W3, b3):
    B, S, D = x.shape
    xf = x.reshape(B * S, D)

    vmem = pl.BlockSpec(memory_space=pltpu.VMEM)
    smem = pl.BlockSpec(memory_space=pltpu.SMEM)

    out_shape = (
        jax.ShapeDtypeStruct((B, _MAX_STATES, state_bank.shape[1]), jnp.float32),
        jax.ShapeDtypeStruct((B, 16), jnp.int32),
    )
    return pl.pallas_call(
        _make_main_body(B, S, D),
        in_specs=[pl.BlockSpec(memory_space=pl.ANY),
                  vmem, vmem, vmem, vmem, vmem, vmem, smem],
        out_specs=(vmem, vmem),
        out_shape=out_shape,
        scratch_shapes=[
            pltpu.VMEM((_NBUF, _CHUNK, D), jnp.float32),
            pltpu.SemaphoreType.DMA((_NBUF,)),
        ],
        compiler_params=pltpu.CompilerParams(
            vmem_limit_bytes=100 * 1024 * 1024),
    )(xf, state_bank, W1, b1, W2, b2, W3, b3)


def _combine_body(rank_ref, ns_ref, mask_ref):
    B = ns_ref.shape[0]
    reps = _MAX_STATES // ns_ref.shape[1]
    ns64 = jnp.concatenate([ns_ref[:, :]] * reps, axis=1)
    rank4 = jnp.broadcast_to(rank_ref[:, :], (B, _MAX_STATES))
    mask_ref[...] = rank4 < ns64


def _combine_call(rankT, ns2):
    vmem = pl.BlockSpec(memory_space=pltpu.VMEM)
    return pl.pallas_call(
        _combine_body,
        in_specs=[vmem, vmem],
        out_specs=vmem,
        out_shape=jax.ShapeDtypeStruct((ns2.shape[0], _MAX_STATES), jnp.bool_),
    )(rankT, ns2)


def kernel(x, state_bank, W1, b1, W2, b2, W3, b3, Wi1, bi1, Wi2, bi2,
           temperature):
    imp2d = _imp_call(state_bank, Wi1, bi1, Wi2, bi2, temperature)
    rankT = _sc_rank_call(imp2d)
    alloc, ns2 = _main_call(x, state_bank, W1, b1, W2, b2, W3, b3)
    mask = _combine_call(rankT, ns2)
    return alloc, mask


# speedup vs baseline: 1.0161x; 1.0017x over previous
import functools

import jax
import jax.numpy as jnp
from jax import lax
from jax.experimental import pallas as pl
from jax.experimental.pallas import tpu as pltpu
from jax.experimental.pallas import tpu_sc as plsc

_MIN_STATES = 4
_MAX_STATES = 64
_CHUNK = 1024
_NBUF = 4


def _imp_body(sb_ref, Wi1_ref, bi1_ref, Wi2_ref, bi2_ref, temp_ref, imp_ref):
    def dot_t(a, w):
        return lax.dot_general(a, w, (((1,), (1,)), ((), ())),
                               preferred_element_type=jnp.float32)

    hi = jax.nn.relu(dot_t(sb_ref[:, :], Wi1_ref[:, :]) + bi1_ref[:][None, :])
    logits = dot_t(hi, jnp.broadcast_to(Wi2_ref[:, :], (16, Wi2_ref.shape[1])))
    logits = logits + bi2_ref[0]
    temp = jnp.maximum(jnp.abs(temp_ref[0]), 0.1)
    imp_ref[...] = jax.nn.softmax(logits / temp, axis=0)


def _imp_call(state_bank, Wi1, bi1, Wi2, bi2, temperature):
    vmem = pl.BlockSpec(memory_space=pltpu.VMEM)
    smem = pl.BlockSpec(memory_space=pltpu.SMEM)
    return pl.pallas_call(
        _imp_body,
        in_specs=[vmem, vmem, vmem, vmem, smem, smem],
        out_specs=vmem,
        out_shape=jax.ShapeDtypeStruct((_MAX_STATES, 16), jnp.float32),
    )(state_bank, Wi1, bi1, Wi2, bi2, temperature)


def _sc_rank_call(imp2d):
    mesh = plsc.VectorSubcoreMesh(core_axis_name="c", subcore_axis_name="s",
                                  num_cores=1)
    n_vregs = _MAX_STATES // 16

    @functools.partial(
        pl.kernel,
        mesh=mesh,
        out_type=jax.ShapeDtypeStruct((1, _MAX_STATES), jnp.int32),
        scratch_types=[
            pltpu.VMEM((_MAX_STATES, 16), jnp.float32),
            pltpu.VMEM((1, _MAX_STATES), jnp.int32),
        ],
    )
    def k(imp_hbm, rank_hbm, imp_v, rank_v):
        wid = lax.axis_index("s") + lax.axis_index("c")

        @pl.when(wid == 0)
        def _():
            pltpu.sync_copy(imp_hbm, imp_v)
            iota = lax.iota(jnp.int32, 16)

            def build(a):
                def lbody(l, v):
                    return jnp.where(iota == l, imp_v[16 * a + l, :], v)
                return lax.fori_loop(1, 16, lbody, imp_v[16 * a, :])

            vs = [build(a) for a in range(n_vregs)]
            gids = [iota + 16 * a for a in range(n_vregs)]

            def jbody(j, ranks):
                vj = imp_v[j, :]
                out = []
                for a in range(n_vregs):
                    cmp = (vj > vs[a]) | ((vj == vs[a]) & (j < gids[a]))
                    out.append(ranks[a] + jnp.where(cmp, 1, 0).astype(jnp.int32))
                return tuple(out)

            zero = jnp.zeros((16,), jnp.int32)
            ranks = lax.fori_loop(0, _MAX_STATES, jbody,
                                  tuple(zero for _ in range(n_vregs)))
            for a in range(n_vregs):
                rank_v[0, pl.ds(16 * a, 16)] = ranks[a]
            pltpu.sync_copy(rank_v, rank_hbm)

    return k(imp2d)


def _make_main_body(B, S, D):
    nchunks = (B * S) // _CHUNK
    chunks_per_batch = S // _CHUNK

    def body(xf_ref, sb_ref, W1_ref, b1_ref, W2_ref, b2_ref, W3_ref, b3_ref,
             alloc_ref, ns_ref, buf_ref, sems):
        def start(i, slot):
            pltpu.make_async_copy(
                xf_ref.at[pl.ds(i * _CHUNK, _CHUNK), :],
                buf_ref.at[slot], sems.at[slot]).start()

        def wait(slot):
            pltpu.make_async_copy(
                xf_ref.at[pl.ds(0, _CHUNK), :],
                buf_ref.at[slot], sems.at[slot]).wait()

        for slot in range(_NBUF):
            start(slot, slot)

        rows8 = jax.lax.broadcasted_iota(jnp.int32, (8, 1), 0)

        def round_body(r, acc):
            for slot in range(_NBUF):
                i = r * _NBUF + slot
                wait(slot)

                @pl.when(i + _NBUF < nchunks)
                def _():
                    start(i + _NBUF, slot)

                partial = jnp.sum(buf_ref[slot], axis=0)
                onehot = (rows8 == i // chunks_per_batch).astype(jnp.float32)
                acc = acc + onehot * partial[None, :]
            return acc

        acc = lax.fori_loop(0, nchunks // _NBUF, round_body,
                            jnp.zeros((8, D), jnp.float32))

        def dot_t(a, w):
            return lax.dot_general(a, w, (((1,), (1,)), ((), ())),
                                   preferred_element_type=jnp.float32)

        pooled = acc[:B, :] * (1.0 / S)
        h = jax.nn.relu(dot_t(pooled, W1_ref[:, :]) + b1_ref[:][None, :])
        h = jax.nn.relu(dot_t(h, W2_ref[:, :]) + b2_ref[:][None, :])
        z = dot_t(h, jnp.broadcast_to(W3_ref[:, :], (16, W3_ref.shape[1])))
        complexity = jax.nn.sigmoid(z + b3_ref[0])
        ns_ref[...] = jnp.clip(
            jnp.round(_MIN_STATES + complexity * (_MAX_STATES - _MIN_STATES)),
            _MIN_STATES, _MAX_STATES).astype(jnp.int32)

        alloc_ref[...] = jnp.broadcast_to(sb_ref[:, :][None, :, :],
                                          alloc_ref.shape)

    return body


def _main_call(x, state_bank, W1, b1, W2, b2, W3, b3):
    B, S, D = x.shape
    xf = x.reshape(B * S, D)

    vmem = pl.BlockSpec(memory_space=pltpu.VMEM)
    smem = pl.BlockSpec(memory_space=pltpu.SMEM)

    out_shape = (
        jax.ShapeDtypeStruct((B, _MAX_STATES, state_bank.shape[1]), jnp.float32),
        jax.ShapeDtypeStruct((B, 16), jnp.int32),
    )
    return pl.pallas_call(
        _make_main_body(B, S, D),
        in_specs=[pl.BlockSpec(memory_space=pl.ANY),
                  vmem, vmem, vmem, vmem, vmem, vmem, smem],
        out_specs=(vmem, vmem),
        out_shape=out_shape,
        scratch_shapes=[
            pltpu.VMEM((_NBUF, _CHUNK, D), jnp.float32),
            pltpu.SemaphoreType.DMA((_NBUF,)),
        ],
        compiler_params=pltpu.CompilerParams(
            vmem_limit_bytes=100 * 1024 * 1024),
    )(xf, state_bank, W1, b1, W2, b2, W3, b3)


def _combine_body(rank_ref, ns_ref, mask_ref):
    B = ns_ref.shape[0]
    reps = _MAX_STATES // ns_ref.shape[1]
    ns64 = jnp.concatenate([ns_ref[:, :]] * reps, axis=1)
    rank4 = jnp.broadcast_to(rank_ref[:, :], (B, _MAX_STATES))
    mask_ref[...] = rank4 < ns64


def _combine_call(rankT, ns2):
    vmem = pl.BlockSpec(memory_space=pltpu.VMEM)
    return pl.pallas_call(
        _combine_body,
        in_specs=[vmem, vmem],
        out_specs=vmem,
        out_shape=jax.ShapeDtypeStruct((ns2.shape[0], _MAX_STATES), jnp.bool_),
    )(rankT, ns2)


def kernel(x, state_bank, W1, b1, W2, b2, W3, b3, Wi1, bi1, Wi2, bi2,
           temperature):
    imp2d = _imp_call(state_bank, Wi1, bi1, Wi2, bi2, temperature)
    rankT = _sc_rank_call(imp2d)
    alloc, ns2 = _main_call(x, state_bank, W1, b1, W2, b2, W3, b3)
    mask = _combine_call(rankT, ns2)
    return alloc, mask
